# baseline (speedup 1.0000x reference)
import functools

import jax
import jax.numpy as jnp
import numpy as np
from jax import lax
from jax.experimental import pallas as pl
from jax.experimental.pallas import tpu as pltpu
from jax.experimental.pallas import tpu_sc as plsc

VOCAB = 100000
EMBED = 64
BATCH = 16384
CTX = 20
NEG = 100
COLS = 128
LE_LAMBDA = 0.01

NC = 2
NS = 16
NW = NC * NS
B_PER_W = BATCH // NW
CHUNK_B = 4
N_CHUNKS = B_PER_W // CHUNK_B

PERM = np.stack([np.arange(16), np.arange(16) + 32,
                 np.arange(16) + 16, np.arange(16) + 48], axis=1).reshape(-1)


def _sc_dots_body(labels_hbm, inlab_hbm, in_w_hbm, out_w_hbm, dots_hbm,
                  inlab_v, in_emb_v, idx_v, rows_v, part_v, dots_v, out_sp,
                  gsem, osem, ssem, hsem):
  wid = lax.axis_index("s") * NC + lax.axis_index("c")
  sid = lax.axis_index("s")
  b0 = wid * B_PER_W

  pltpu.make_async_copy(
      out_w_hbm.at[pl.ds(sid * (VOCAB // NS), VOCAB // NS)],
      out_sp.at[pl.ds(sid * (VOCAB // NS), VOCAB // NS)], ssem).start()

  pltpu.sync_copy(inlab_hbm.at[pl.ds(wid * 4, 4)], inlab_v)
  for j in range(4):
    pltpu.make_async_copy(
        in_w_hbm.at[inlab_v.at[j]],
        in_emb_v.at[pl.ds(j * 128, 128)], gsem).start()
  for j in range(4):
    pltpu.make_async_copy(
        in_w_hbm.at[inlab_v.at[j]],
        in_emb_v.at[pl.ds(j * 128, 128)], gsem).wait()

  iota16x16 = lax.iota(jnp.int32, 16) * 16

  pltpu.make_async_copy(
      out_w_hbm.at[pl.ds(sid * (VOCAB // NS), VOCAB // NS)],
      out_sp.at[pl.ds(sid * (VOCAB // NS), VOCAB // NS)], ssem).wait()
  plsc.subcore_barrier()

  def stage(g, buf):
    pltpu.sync_copy(labels_hbm.at[pl.ds(b0 + g * CHUNK_B, CHUNK_B)],
                    idx_v.at[buf])
    for k in range(CHUNK_B):
      src = out_sp if k % 2 == 0 else out_w_hbm
      sem = gsem if k % 2 == 0 else hsem
      pltpu.make_async_copy(
          src.at[idx_v.at[buf, k]],
          rows_v.at[buf, pl.ds(k * COLS, COLS)], sem).start()

  stage(0, 0)

  def chunk_body(g, carry):
    buf = lax.rem(g, 2)
    nbuf = lax.rem(g + 1, 2)

    @pl.when(g + 1 < N_CHUNKS)
    def _():
      stage(g + 1, nbuf)

    for k in range(CHUNK_B):
      src = out_sp if k % 2 == 0 else out_w_hbm
      sem = gsem if k % 2 == 0 else hsem
      pltpu.make_async_copy(
          src.at[idx_v.at[buf, k]],
          rows_v.at[buf, pl.ds(k * COLS, COLS)], sem).wait()

    @pl.when(g >= 2)
    def _():
      pltpu.make_async_copy(
          dots_v.at[buf],
          dots_hbm.at[pl.ds(b0 + (g - 2) * CHUNK_B, CHUNK_B)], osem).wait()

    for bb in range(CHUNK_B):
      b_loc = g * CHUNK_B + bb
      ea, eb = plsc.unpack(in_emb_v[b_loc, :],
                           format=plsc.PackFormat.INTERLEAVED,
                           preferred_element_type=jnp.bfloat16)
      e0, e1 = plsc.unpack(ea, format=plsc.PackFormat.INTERLEAVED,
                           preferred_element_type=jnp.float32)
      e2, e3 = plsc.unpack(eb, format=plsc.PackFormat.INTERLEAVED,
                           preferred_element_type=jnp.float32)

      def group_body(g2, c2):
        base = bb * COLS + g2 * 16
        for l in range(16):
          r = base + l
          ha, hb = plsc.unpack(rows_v[buf, r, :],
                               format=plsc.PackFormat.INTERLEAVED,
                               preferred_element_type=jnp.bfloat16)
          v0, v1 = plsc.unpack(ha, format=plsc.PackFormat.INTERLEAVED,
                               preferred_element_type=jnp.float32)
          v2, v3 = plsc.unpack(hb, format=plsc.PackFormat.INTERLEAVED,
                               preferred_element_type=jnp.float32)
          p = (v0 * e0 + v1 * e1) + (v2 * e2 + v3 * e3)
          part_v[pl.ds(l * 16, 16)] = p
        terms = [plsc.load_gather(part_v, [iota16x16 + e]) for e in range(16)]
        while len(terms) > 1:
          terms = [terms[i] + terms[i + 1] for i in range(0, len(terms), 2)]
        dots_v[buf, bb, pl.ds(g2 * 16, 16)] = terms[0]
        return c2

      lax.fori_loop(0, COLS // 16, group_body, 0, unroll=2)

    pltpu.make_async_copy(
        dots_v.at[buf],
        dots_hbm.at[pl.ds(b0 + g * CHUNK_B, CHUNK_B)], osem).start()
    return carry

  lax.fori_loop(0, N_CHUNKS, chunk_body, 0, unroll=False)

  for g in (N_CHUNKS - 2, N_CHUNKS - 1):
    pltpu.make_async_copy(
        dots_v.at[g % 2],
        dots_hbm.at[pl.ds(b0 + g * CHUNK_B, CHUNK_B)], osem).wait()


_sc_dots = functools.partial(
    pl.kernel,
    out_type=jax.ShapeDtypeStruct((BATCH, COLS), jnp.float32),
    mesh=plsc.VectorSubcoreMesh(core_axis_name="c", subcore_axis_name="s"),
    compiler_params=pltpu.CompilerParams(
        needs_layout_passes=False, use_tc_tiling_on_sc=False),
    scratch_types=[
        pltpu.VMEM((4, 128), jnp.int32),
        pltpu.VMEM((B_PER_W, EMBED), jnp.float8_e4m3fn),
        pltpu.VMEM((2, CHUNK_B, COLS), jnp.int32),
        pltpu.VMEM((2, CHUNK_B * COLS, EMBED), jnp.float8_e4m3fn),
        pltpu.VMEM((256,), jnp.float32),
        pltpu.VMEM((2, CHUNK_B, COLS), jnp.float32),
        pltpu.VMEM_SHARED((VOCAB, EMBED), jnp.float8_e4m3fn),
        pltpu.SemaphoreType.DMA,
        pltpu.SemaphoreType.DMA,
        pltpu.SemaphoreType.DMA,
        pltpu.SemaphoreType.DMA,
    ],
)(_sc_dots_body)


def _tc_loss_body(dots_ref, even_ref, odd_ref, out1_ref, out2_ref):
  x = dots_ref[...]
  col = lax.broadcasted_iota(jnp.int32, (BATCH, COLS), 1)
  a = jnp.where(col < CTX, x, -x)
  ls = jnp.minimum(a, 0.0) - jnp.log1p(jnp.exp(-jnp.abs(a)))
  s = jnp.sum(jnp.where(col < CTX + NEG, ls, 0.0))
  d = even_ref[...] - odd_ref[...]
  norms = jnp.sqrt(jnp.sum(d * d, axis=1))
  le = 0.5 * jnp.sum(norms) ** 2 * LE_LAMBDA
  out1_ref[...] = jnp.reshape(-(s / BATCH) + le, (1, 1))
  out2_ref[...] = jnp.reshape(le, (1, 1))


def _tc_loss(dots, even, odd):
  return pl.pallas_call(
      _tc_loss_body,
      out_shape=[jax.ShapeDtypeStruct((1, 1), jnp.float32),
                 jax.ShapeDtypeStruct((1, 1), jnp.float32)],
  )(dots, even, odd)


def kernel(input_labels, pos_labels, neg_labels, in_embed_w, out_embed_w):
  pos = pos_labels.astype(jnp.int32)
  neg = neg_labels.astype(jnp.int32)
  labels = jnp.concatenate(
      [pos, neg, jnp.zeros((BATCH, COLS - CTX - NEG), jnp.int32)], axis=1)
  inlab2d = input_labels.astype(jnp.int32).reshape(BATCH // 128, 128)
  in_w_f8 = in_embed_w[:, PERM].astype(jnp.float8_e4m3fn)
  out_w_f8 = out_embed_w[:, PERM].astype(jnp.float8_e4m3fn)
  dots = _sc_dots(labels, inlab2d, in_w_f8, out_w_f8)
  even = in_embed_w[0:64:2]
  odd = in_embed_w[1:64:2]
  loss_combined, loss_le = _tc_loss(dots, even, odd)
  return (loss_combined[0, 0], loss_le[0, 0])


# speedup vs baseline: 1.1379x; 1.1379x over previous
import functools

import jax
import jax.numpy as jnp
import numpy as np
from jax import lax
from jax.experimental import pallas as pl
from jax.experimental.pallas import tpu as pltpu
from jax.experimental.pallas import tpu_sc as plsc

VOCAB = 100000
EMBED = 64
BATCH = 16384
CTX = 20
NEG = 100
COLS = 128
LE_LAMBDA = 0.01

NC = 2
NS = 16
NW = NC * NS
B_PER_W = BATCH // NW
CHUNK_B = 4
N_CHUNKS = B_PER_W // CHUNK_B

PERM = np.stack([np.arange(16), np.arange(16) + 32,
                 np.arange(16) + 16, np.arange(16) + 48], axis=1).reshape(-1)


def _sc_dots_body(labels_hbm, inlab_hbm, in_w_hbm, out_w_hbm, dots_hbm,
                  inlab_v, in_emb_v, idx_v, rows_v, part_v, dots_v,
                  gsem, osem, isem):
  wid = lax.axis_index("s") * NC + lax.axis_index("c")
  b0 = wid * B_PER_W

  pltpu.sync_copy(inlab_hbm.at[pl.ds(wid * 4, 4)], inlab_v)
  for j in range(4):
    pltpu.make_async_copy(
        in_w_hbm.at[inlab_v.at[j]],
        in_emb_v.at[pl.ds(j * 128, 128)], gsem).start()
  for j in range(4):
    pltpu.make_async_copy(
        in_w_hbm.at[inlab_v.at[j]],
        in_emb_v.at[pl.ds(j * 128, 128)], gsem).wait()

  iota16x16 = lax.iota(jnp.int32, 16) * 16

  def idx_copy(g, ib):
    return pltpu.make_async_copy(
        labels_hbm.at[pl.ds(b0 + g * CHUNK_B, CHUNK_B)], idx_v.at[ib], isem)

  def fire(g, ib, rb):
    for k in range(CHUNK_B):
      pltpu.make_async_copy(
          out_w_hbm.at[idx_v.at[ib, k]],
          rows_v.at[rb, pl.ds(k * COLS, COLS)], gsem).start()

  pltpu.sync_copy(labels_hbm.at[pl.ds(b0, CHUNK_B)], idx_v.at[0])
  fire(0, 0, 0)
  idx_copy(1, 1).start()

  def chunk_body(g, carry):
    buf = lax.rem(g, 2)
    nbuf = lax.rem(g + 1, 2)

    @pl.when(g + 1 < N_CHUNKS)
    def _():
      idx_copy(g + 1, lax.rem(g + 1, 3)).wait()
      fire(g + 1, lax.rem(g + 1, 3), nbuf)

    @pl.when(g + 2 < N_CHUNKS)
    def _():
      idx_copy(g + 2, lax.rem(g + 2, 3)).start()

    ib = lax.rem(g, 3)
    for k in range(CHUNK_B):
      pltpu.make_async_copy(
          out_w_hbm.at[idx_v.at[ib, k]],
          rows_v.at[buf, pl.ds(k * COLS, COLS)], gsem).wait()

    @pl.when(g >= 2)
    def _():
      pltpu.make_async_copy(
          dots_v.at[buf],
          dots_hbm.at[pl.ds(b0 + (g - 2) * CHUNK_B, CHUNK_B)], osem).wait()

    for bb in range(CHUNK_B):
      b_loc = g * CHUNK_B + bb
      e0 = in_emb_v[b_loc, pl.ds(0, 16)]
      e1 = in_emb_v[b_loc, pl.ds(16, 16)]
      e2 = in_emb_v[b_loc, pl.ds(32, 16)]
      e3 = in_emb_v[b_loc, pl.ds(48, 16)]

      def group_body(g2, c2):
        base = bb * COLS + g2 * 16
        for l in range(16):
          r = base + l
          ha, hb = plsc.unpack(rows_v[buf, r, :],
                               format=plsc.PackFormat.INTERLEAVED,
                               preferred_element_type=jnp.bfloat16)
          v0, v1 = plsc.unpack(ha, format=plsc.PackFormat.INTERLEAVED,
                               preferred_element_type=jnp.float32)
          v2, v3 = plsc.unpack(hb, format=plsc.PackFormat.INTERLEAVED,
                               preferred_element_type=jnp.float32)
          p = (v0 * e0 + v1 * e1) + (v2 * e2 + v3 * e3)
          part_v[pl.ds(l * 16, 16)] = p
        terms = [plsc.load_gather(part_v, [iota16x16 + e]) for e in range(16)]
        while len(terms) > 1:
          terms = [terms[i] + terms[i + 1] for i in range(0, len(terms), 2)]
        dots_v[buf, bb, pl.ds(g2 * 16, 16)] = terms[0]
        return c2

      lax.fori_loop(0, COLS // 16, group_body, 0, unroll=2)

    pltpu.make_async_copy(
        dots_v.at[buf],
        dots_hbm.at[pl.ds(b0 + g * CHUNK_B, CHUNK_B)], osem).start()
    return carry

  lax.fori_loop(0, N_CHUNKS, chunk_body, 0, unroll=False)

  for g in (N_CHUNKS - 2, N_CHUNKS - 1):
    pltpu.make_async_copy(
        dots_v.at[g % 2],
        dots_hbm.at[pl.ds(b0 + g * CHUNK_B, CHUNK_B)], osem).wait()


_sc_dots = functools.partial(
    pl.kernel,
    out_type=jax.ShapeDtypeStruct((BATCH, COLS), jnp.float32),
    mesh=plsc.VectorSubcoreMesh(core_axis_name="c", subcore_axis_name="s"),
    compiler_params=pltpu.CompilerParams(
        needs_layout_passes=False, use_tc_tiling_on_sc=False),
    scratch_types=[
        pltpu.VMEM((4, 128), jnp.int32),
        pltpu.VMEM((B_PER_W, EMBED), jnp.float32),
        pltpu.VMEM((3, CHUNK_B, COLS), jnp.int32),
        pltpu.VMEM((2, CHUNK_B * COLS, EMBED), jnp.float8_e4m3fn),
        pltpu.VMEM((256,), jnp.float32),
        pltpu.VMEM((2, CHUNK_B, COLS), jnp.float32),
        pltpu.SemaphoreType.DMA,
        pltpu.SemaphoreType.DMA,
        pltpu.SemaphoreType.DMA,
    ],
)(_sc_dots_body)


def _tc_loss_body(dots_ref, even_ref, odd_ref, out1_ref, out2_ref):
  x = dots_ref[...]
  col = lax.broadcasted_iota(jnp.int32, (BATCH, COLS), 1)
  a = jnp.where(col < CTX, x, -x)
  ls = jnp.minimum(a, 0.0) - jnp.log1p(jnp.exp(-jnp.abs(a)))
  s = jnp.sum(jnp.where(col < CTX + NEG, ls, 0.0))
  d = even_ref[...] - odd_ref[...]
  norms = jnp.sqrt(jnp.sum(d * d, axis=1))
  le = 0.5 * jnp.sum(norms) ** 2 * LE_LAMBDA
  out1_ref[...] = jnp.reshape(-(s / BATCH) + le, (1, 1))
  out2_ref[...] = jnp.reshape(le, (1, 1))


def _tc_loss(dots, even, odd):
  return pl.pallas_call(
      _tc_loss_body,
      out_shape=[jax.ShapeDtypeStruct((1, 1), jnp.float32),
                 jax.ShapeDtypeStruct((1, 1), jnp.float32)],
  )(dots, even, odd)


def kernel(input_labels, pos_labels, neg_labels, in_embed_w, out_embed_w):
  pos = pos_labels.astype(jnp.int32)
  neg = neg_labels.astype(jnp.int32)
  labels = jnp.concatenate(
      [pos, neg, jnp.zeros((BATCH, COLS - CTX - NEG), jnp.int32)], axis=1)
  inlab2d = input_labels.astype(jnp.int32).reshape(BATCH // 128, 128)
  out_w_f8 = out_embed_w[:, PERM].astype(jnp.float8_e4m3fn)
  dots = _sc_dots(labels, inlab2d, in_embed_w, out_w_f8)
  even = in_embed_w[0:64:2]
  odd = in_embed_w[1:64:2]
  loss_combined, loss_le = _tc_loss(dots, even, odd)
  return (loss_combined[0, 0], loss_le[0, 0])
